# Initial kernel scaffold; baseline (speedup 1.0000x reference)
import jax
import jax.numpy as jnp
from jax.experimental import pallas as pl


def kernel(h, edge_index, norm, edge_weight, W, b):
    raise NotImplementedError("write your pallas kernel here")


# speedup vs baseline: 5.3799x; 5.3799x over previous
import dataclasses
import functools

import jax
import jax.numpy as jnp
from jax import lax
from jax.experimental import pallas as pl
from jax.experimental.pallas import tpu as pltpu
from jax.experimental.pallas import tpu_sc as plsc

N = 10000
D = 128
E = 320000

NC = 2
NS = 16
LANES = 16
NW = NC * NS

K = 128
EPW = 10240
EP = NW * EPW

N_PAD = 10240
ROWS_PER_TILE = N_PAD // NS
ZCHUNK = 128

_MESH = plsc.VectorSubcoreMesh(
    core_axis_name="c", subcore_axis_name="s", num_cores=NC, num_subcores=NS)

_SC_PARAMS = pltpu.CompilerParams()
if "needs_layout_passes" in pltpu.CompilerParams.__dataclass_fields__:
    _SC_PARAMS = dataclasses.replace(_SC_PARAMS, needs_layout_passes=False)


def _gemm_body(h_ref, w_ref, norm_ref, g_ref):
    g_ref[...] = jnp.dot(
        h_ref[...], w_ref[...], preferred_element_type=jnp.float32
    ) * norm_ref[...]


def _compute_g(h, W, norm):
    M = 1000
    return pl.pallas_call(
        _gemm_body,
        grid=(N // M,),
        in_specs=[
            pl.BlockSpec((M, D), lambda i: (i, i * 0)),
            pl.BlockSpec((D, D), lambda i: (i * 0, i * 0)),
            pl.BlockSpec((M, 1), lambda i: (i, i * 0)),
        ],
        out_specs=pl.BlockSpec((M, D), lambda i: (i, i * 0)),
        out_shape=jax.ShapeDtypeStruct((N, D), jnp.float32),
    )(h, W, norm)


def _floop(n, body):
    def wrapped(i, carry):
        body(i)
        return carry
    lax.fori_loop(jnp.int32(0), jnp.int32(n), wrapped, None)


def _sc_body(g_hbm, src_hbm, dst_hbm, w_hbm, out_hbm,
             src_v, dst_v, w_v, rows_v, acc, gsem):
    i32 = jnp.int32
    cid = lax.axis_index("c").astype(i32)
    sid = lax.axis_index("s").astype(i32)
    wid = sid * i32(NC) + cid

    zero16 = jnp.zeros((LANES,), jnp.float32)

    def _zero_row(r):
        for c in range(D // LANES):
            rows_v[r, pl.ds(c * LANES, LANES)] = zero16
    _floop(K, _zero_row)

    tile_base = sid * i32(ROWS_PER_TILE)
    for j in range(ROWS_PER_TILE // ZCHUNK):
        pltpu.sync_copy(
            rows_v.at[pl.ds(0, ZCHUNK)],
            acc.at[pl.ds(tile_base + i32(j * ZCHUNK), ZCHUNK)],
        )

    plsc.subcore_barrier()

    ebase = wid * i32(EPW)

    def _chunk(i):
        base = ebase + i * i32(K)
        pltpu.sync_copy(src_hbm.at[pl.ds(base, K)], src_v)
        pltpu.sync_copy(dst_hbm.at[pl.ds(base, K)], dst_v)
        pltpu.sync_copy(w_hbm.at[pl.ds(base, K)], w_v)
        pltpu.async_copy(g_hbm.at[src_v], rows_v, gsem).wait()

        def _scale(e):
            eidx = jnp.broadcast_to(e, (LANES,))
            wv = plsc.load_gather(w_v, [eidx])
            for c in range(D // LANES):
                sl = pl.ds(c * LANES, LANES)
                rows_v[e, sl] = rows_v[e, sl] * wv
        _floop(K, _scale)

        pltpu.sync_copy(rows_v, acc.at[dst_v], add=True)

    _floop(EPW // K, _chunk)

    plsc.subcore_barrier()

    for j in range(ROWS_PER_TILE // ZCHUNK):
        sl = pl.ds(tile_base + i32(j * ZCHUNK), ZCHUNK)
        pltpu.sync_copy(acc.at[sl], out_hbm.at[cid, sl])


@functools.partial(
    pl.kernel,
    out_type=jax.ShapeDtypeStruct((NC, N_PAD, D), jnp.float32),
    mesh=_MESH,
    scratch_types=[
        pltpu.VMEM((K,), jnp.int32),
        pltpu.VMEM((K,), jnp.int32),
        pltpu.VMEM((K,), jnp.float32),
        pltpu.VMEM((K, D), jnp.float32),
        pltpu.VMEM_SHARED((N_PAD, D), jnp.float32),
        pltpu.SemaphoreType.DMA,
    ],
    compiler_params=_SC_PARAMS,
)
def _sc_scatter(g_hbm, src_hbm, dst_hbm, w_hbm, out_hbm,
                src_v, dst_v, w_v, rows_v, acc, gsem):
    _sc_body(g_hbm, src_hbm, dst_hbm, w_hbm, out_hbm,
             src_v, dst_v, w_v, rows_v, acc, gsem)


def _combine_body(p_ref, norm_ref, b_ref, o_ref):
    o_ref[...] = (p_ref[0] + p_ref[1]) * norm_ref[...] + b_ref[...]


def _combine(partials, norm, b2d):
    M = 1000
    return pl.pallas_call(
        _combine_body,
        grid=(N // M,),
        in_specs=[
            pl.BlockSpec((NC, M, D), lambda i: (i * 0, i, i * 0)),
            pl.BlockSpec((M, 1), lambda i: (i, i * 0)),
            pl.BlockSpec((1, D), lambda i: (i * 0, i * 0)),
        ],
        out_specs=pl.BlockSpec((M, D), lambda i: (i, i * 0)),
        out_shape=jax.ShapeDtypeStruct((N, D), jnp.float32),
    )(partials, norm, b2d)


def kernel(h, edge_index, norm, edge_weight, W, b):
    src = edge_index[0].astype(jnp.int32)
    dst = edge_index[1].astype(jnp.int32)
    w = edge_weight.reshape(E).astype(jnp.float32)

    pad = EP - E
    zi = jnp.zeros((pad,), jnp.int32)
    src = jnp.concatenate([src, zi])
    dst = jnp.concatenate([dst, zi])
    w = jnp.concatenate([w, jnp.zeros((pad,), jnp.float32)])

    g = _compute_g(h, W, norm)
    partials = _sc_scatter(g, src, dst, w)
    return _combine(partials, norm, b.reshape(1, D))


# speedup vs baseline: 6.0376x; 1.1222x over previous
import dataclasses
import functools

import jax
import jax.numpy as jnp
from jax import lax
from jax.experimental import pallas as pl
from jax.experimental.pallas import tpu as pltpu
from jax.experimental.pallas import tpu_sc as plsc

N = 10000
D = 128
E = 320000

NC = 2
NS = 16
LANES = 16
NW = NC * NS

K = 128
EPW = 10240
EP = NW * EPW

N_PAD = 10240
ROWS_PER_TILE = N_PAD // NS
ZCHUNK = 128

_MESH = plsc.VectorSubcoreMesh(
    core_axis_name="c", subcore_axis_name="s", num_cores=NC, num_subcores=NS)

_SC_PARAMS = pltpu.CompilerParams()
if "needs_layout_passes" in pltpu.CompilerParams.__dataclass_fields__:
    _SC_PARAMS = dataclasses.replace(_SC_PARAMS, needs_layout_passes=False)


def _gemm_body(h_ref, w_ref, norm_ref, g_ref):
    g_ref[...] = jnp.dot(
        h_ref[...], w_ref[...], preferred_element_type=jnp.float32
    ) * norm_ref[...]


def _compute_g(h, W, norm):
    M = 1000
    return pl.pallas_call(
        _gemm_body,
        grid=(N // M,),
        in_specs=[
            pl.BlockSpec((M, D), lambda i: (i, i * 0)),
            pl.BlockSpec((D, D), lambda i: (i * 0, i * 0)),
            pl.BlockSpec((M, 1), lambda i: (i, i * 0)),
        ],
        out_specs=pl.BlockSpec((M, D), lambda i: (i, i * 0)),
        out_shape=jax.ShapeDtypeStruct((N, D), jnp.float32),
    )(h, W, norm)


NCHUNK = EPW // K


def _floop(n, body, unroll=None):
    def wrapped(i, carry):
        body(i)
        return carry
    lax.fori_loop(jnp.int32(0), jnp.int32(n), wrapped, None, unroll=unroll)


def _sc_body(g_hbm, src_hbm, dst_hbm, w_hbm, out_hbm,
             src_all, dstb, wb, rows0, rows1, acc,
             gsem0, gsem1, dsem0, dsem1, wsem0, wsem1):
    i32 = jnp.int32
    cid = lax.axis_index("c").astype(i32)
    sid = lax.axis_index("s").astype(i32)
    wid = sid * i32(NC) + cid

    zero16 = jnp.zeros((LANES,), jnp.float32)

    def _zero_row(r):
        for c in range(D // LANES):
            rows0[r, pl.ds(c * LANES, LANES)] = zero16
    _floop(K, _zero_row)

    tile_base = sid * i32(ROWS_PER_TILE)
    for j in range(ROWS_PER_TILE // ZCHUNK):
        pltpu.sync_copy(
            rows0.at[pl.ds(0, ZCHUNK)],
            acc.at[pl.ds(tile_base + i32(j * ZCHUNK), ZCHUNK)],
        )

    pltpu.sync_copy(src_hbm.at[wid], src_all)

    plsc.subcore_barrier()

    dsems = (dsem0, dsem1)
    wsems = (wsem0, wsem1)

    def _idx_prefetch(i, p):
        pltpu.async_copy(dst_hbm.at[wid, i], dstb.at[i32(p)], dsems[p])
        pltpu.async_copy(w_hbm.at[wid, i], wb.at[pl.ds(p * K, K)], wsems[p])

    def _idx_wait(p):
        z = i32(0)
        pltpu.make_async_copy(
            dst_hbm.at[wid, z], dstb.at[i32(p)], dsems[p]).wait()
        pltpu.make_async_copy(
            w_hbm.at[wid, z], wb.at[pl.ds(p * K, K)], wsems[p]).wait()

    def _gather_start(i, rows, sem):
        pltpu.async_copy(g_hbm.at[src_all.at[i]], rows, sem)

    def _gather_wait(rows, sem):
        pltpu.make_async_copy(
            g_hbm.at[src_all.at[jnp.int32(0)]], rows, sem).wait()

    def _scale(rows, p):
        base = i32(p * K)
        U = 4
        def body(eu):
            e0 = eu * i32(U)
            for u in range(U):
                e = e0 + i32(u)
                eidx = jnp.broadcast_to(base + e, (LANES,))
                wv = plsc.load_gather(wb, [eidx])
                for c in range(D // LANES):
                    sl = pl.ds(c * LANES, LANES)
                    rows[e, sl] = rows[e, sl] * wv
        _floop(K // U, body)

    _idx_prefetch(i32(0), 0)
    _idx_prefetch(i32(1), 1)
    _gather_start(i32(0), rows0, gsem0)

    def _pair(j):
        i0 = j * i32(2)
        i1 = i0 + i32(1)
        _gather_start(i1, rows1, gsem1)
        _gather_wait(rows0, gsem0)
        _idx_wait(0)
        _scale(rows0, 0)
        pltpu.sync_copy(rows0, acc.at[dstb.at[i32(0)]], add=True)
        _idx_prefetch(i0 + i32(2), 0)
        _gather_start(i0 + i32(2), rows0, gsem0)
        _gather_wait(rows1, gsem1)
        _idx_wait(1)
        _scale(rows1, 1)
        pltpu.sync_copy(rows1, acc.at[dstb.at[i32(1)]], add=True)
        _idx_prefetch(i1 + i32(2), 1)

    _floop(NCHUNK // 2, _pair)
    _gather_wait(rows0, gsem0)
    _idx_wait(0)
    _idx_wait(1)

    plsc.subcore_barrier()

    for j in range(ROWS_PER_TILE // ZCHUNK):
        sl = pl.ds(tile_base + i32(j * ZCHUNK), ZCHUNK)
        pltpu.sync_copy(acc.at[sl], out_hbm.at[cid, sl])


@functools.partial(
    pl.kernel,
    out_type=jax.ShapeDtypeStruct((NC, N_PAD, D), jnp.float32),
    mesh=_MESH,
    scratch_types=[
        pltpu.VMEM((NCHUNK + 1, K), jnp.int32),
        pltpu.VMEM((2, K), jnp.int32),
        pltpu.VMEM((2 * K,), jnp.float32),
        pltpu.VMEM((K, D), jnp.float32),
        pltpu.VMEM((K, D), jnp.float32),
        pltpu.VMEM_SHARED((N_PAD, D), jnp.float32),
        pltpu.SemaphoreType.DMA,
        pltpu.SemaphoreType.DMA,
        pltpu.SemaphoreType.DMA,
        pltpu.SemaphoreType.DMA,
        pltpu.SemaphoreType.DMA,
        pltpu.SemaphoreType.DMA,
    ],
    compiler_params=_SC_PARAMS,
)
def _sc_scatter(g_hbm, src_hbm, dst_hbm, w_hbm, out_hbm,
                src_all, dstb, wb, rows0, rows1, acc,
                gsem0, gsem1, dsem0, dsem1, wsem0, wsem1):
    _sc_body(g_hbm, src_hbm, dst_hbm, w_hbm, out_hbm,
             src_all, dstb, wb, rows0, rows1, acc,
             gsem0, gsem1, dsem0, dsem1, wsem0, wsem1)


def _combine_body(p_ref, norm_ref, b_ref, o_ref):
    o_ref[...] = (p_ref[0] + p_ref[1]) * norm_ref[...] + b_ref[...]


def _combine(partials, norm, b2d):
    M = 1000
    return pl.pallas_call(
        _combine_body,
        grid=(N // M,),
        in_specs=[
            pl.BlockSpec((NC, M, D), lambda i: (i * 0, i, i * 0)),
            pl.BlockSpec((M, 1), lambda i: (i, i * 0)),
            pl.BlockSpec((1, D), lambda i: (i * 0, i * 0)),
        ],
        out_specs=pl.BlockSpec((M, D), lambda i: (i, i * 0)),
        out_shape=jax.ShapeDtypeStruct((N, D), jnp.float32),
    )(partials, norm, b2d)


def kernel(h, edge_index, norm, edge_weight, W, b):
    src = edge_index[0].astype(jnp.int32)
    dst = edge_index[1].astype(jnp.int32)
    w = edge_weight.reshape(E).astype(jnp.float32)

    pad = EP - E
    zi = jnp.zeros((pad,), jnp.int32)
    src = jnp.concatenate([src, zi]).reshape(NW, NCHUNK, K)
    dst = jnp.concatenate([dst, zi]).reshape(NW, NCHUNK, K)
    w = jnp.concatenate([w, jnp.zeros((pad,), jnp.float32)]).reshape(NW, NCHUNK, K)
    src = jnp.concatenate([src, jnp.zeros((NW, 1, K), jnp.int32)], axis=1)
    dst = jnp.concatenate([dst, jnp.zeros((NW, 2, K), jnp.int32)], axis=1)
    w = jnp.concatenate([w, jnp.zeros((NW, 2, K), jnp.float32)], axis=1)

    g = _compute_g(h, W, norm)
    partials = _sc_scatter(g, src, dst, w)
    return _combine(partials, norm, b.reshape(1, D))
